# Initial kernel scaffold; baseline (speedup 1.0000x reference)
import jax
import jax.numpy as jnp
from jax.experimental import pallas as pl


def kernel(hidden_states, w_router, w1, b1, w2, b2):
    raise NotImplementedError("write your pallas kernel here")


# speedup vs baseline: 1.2949x; 1.2949x over previous
import jax
import jax.numpy as jnp
from jax.experimental import pallas as pl
from jax.experimental.pallas import tpu as pltpu

N_EXP = 8
TOP2_BIG_NEG = -1e30


def _moe_dense_body(flat_ref, wr_ref, w1_ref, b1_ref, w2_ref, b2_ref,
                    out_ref, comb_ref, xbf_ref):
    e = pl.program_id(0)
    t = flat_ref.shape[0]

    @pl.when(e == 0)
    def _init():
        x = flat_ref[...]
        logits = jax.lax.dot_general(
            x, wr_ref[...], (((1,), (0,)), ((), ())),
            preferred_element_type=jnp.float32,
            precision=jax.lax.Precision.HIGHEST)
        eids = jax.lax.broadcasted_iota(jnp.int32, (t, N_EXP), 1)
        m1 = jnp.max(logits, axis=1, keepdims=True)
        i1 = jnp.min(jnp.where(logits == m1, eids, N_EXP), axis=1, keepdims=True)
        l2 = jnp.where(eids == i1, TOP2_BIG_NEG, logits)
        m2 = jnp.max(l2, axis=1, keepdims=True)
        i2 = jnp.min(jnp.where(l2 == m2, eids, N_EXP), axis=1, keepdims=True)
        p = jnp.exp(logits - m1)
        z = jnp.sum(p, axis=1, keepdims=True)
        probs = p / z
        sel = (eids == i1) | (eids == i2)
        comb_ref[...] = jnp.where(sel, probs, 0.0)
        xbf_ref[...] = x.astype(jnp.bfloat16)
        out_ref[...] = x

    xbf = xbf_ref[...]
    h = jnp.dot(xbf, w1_ref[0].astype(jnp.bfloat16),
                preferred_element_type=jnp.float32)
    h = jnp.maximum(h + b1_ref[0], 0.0).astype(jnp.bfloat16)
    y = jnp.dot(h, w2_ref[0].astype(jnp.bfloat16),
                preferred_element_type=jnp.float32)
    y = y + b2_ref[0]
    eids = jax.lax.broadcasted_iota(jnp.int32, (t, N_EXP), 1)
    c = jnp.sum(jnp.where(eids == e, comb_ref[...], 0.0), axis=1, keepdims=True)
    out_ref[...] += c * y


def kernel(hidden_states, w_router, w1, b1, w2, b2):
    b, s, d = hidden_states.shape
    t = b * s
    f = w1.shape[-1]
    flat = hidden_states.reshape(t, d)

    out = pl.pallas_call(
        _moe_dense_body,
        grid=(N_EXP,),
        in_specs=[
            pl.BlockSpec((t, d), lambda e: (0, 0)),
            pl.BlockSpec((d, N_EXP), lambda e: (0, 0)),
            pl.BlockSpec((1, d, f), lambda e: (e, 0, 0)),
            pl.BlockSpec((1, 1, f), lambda e: (e, 0, 0)),
            pl.BlockSpec((1, f, d), lambda e: (e, 0, 0)),
            pl.BlockSpec((1, 1, d), lambda e: (e, 0, 0)),
        ],
        out_specs=pl.BlockSpec((t, d), lambda e: (0, 0)),
        out_shape=jax.ShapeDtypeStruct((t, d), jnp.float32),
        scratch_shapes=[
            pltpu.VMEM((t, N_EXP), jnp.float32),
            pltpu.VMEM((t, d), jnp.bfloat16),
        ],
        compiler_params=pltpu.CompilerParams(
            dimension_semantics=("arbitrary",),
        ),
    )(flat, w_router, w1, b1.reshape(N_EXP, 1, f), w2, b2.reshape(N_EXP, 1, d))
    return out.reshape(b, s, d)


# speedup vs baseline: 1.2977x; 1.0022x over previous
import jax
import jax.numpy as jnp
from jax.experimental import pallas as pl
from jax.experimental.pallas import tpu as pltpu

N_EXP = 8
TOP2_BIG_NEG = -1e30


def _moe_dense_body(flat_ref, wr_ref, w1_ref, b1_ref, w2_ref, b2_ref,
                    out_ref, comb_ref):
    e = pl.program_id(0)
    t = flat_ref.shape[0]

    @pl.when(e == 0)
    def _init():
        x = flat_ref[...]
        logits = jax.lax.dot_general(
            x, wr_ref[...], (((1,), (0,)), ((), ())),
            preferred_element_type=jnp.float32,
            precision=jax.lax.Precision.HIGHEST)
        eids = jax.lax.broadcasted_iota(jnp.int32, (t, N_EXP), 1)
        m1 = jnp.max(logits, axis=1, keepdims=True)
        i1 = jnp.min(jnp.where(logits == m1, eids, N_EXP), axis=1, keepdims=True)
        l2 = jnp.where(eids == i1, TOP2_BIG_NEG, logits)
        m2 = jnp.max(l2, axis=1, keepdims=True)
        i2 = jnp.min(jnp.where(l2 == m2, eids, N_EXP), axis=1, keepdims=True)
        p = jnp.exp(logits - m1)
        z = jnp.sum(p, axis=1, keepdims=True)
        probs = p / z
        sel = (eids == i1) | (eids == i2)
        comb_ref[...] = jnp.where(sel, probs, 0.0)
        out_ref[...] = x

    x = flat_ref[...]
    h = jnp.dot(x, w1_ref[0], preferred_element_type=jnp.float32)
    h = jnp.maximum(h + b1_ref[0], 0.0)
    y = jnp.dot(h, w2_ref[0], preferred_element_type=jnp.float32)
    y = y + b2_ref[0]
    eids = jax.lax.broadcasted_iota(jnp.int32, (t, N_EXP), 1)
    c = jnp.sum(jnp.where(eids == e, comb_ref[...], 0.0), axis=1, keepdims=True)
    out_ref[...] += c * y


def kernel(hidden_states, w_router, w1, b1, w2, b2):
    b, s, d = hidden_states.shape
    t = b * s
    f = w1.shape[-1]
    flat = hidden_states.reshape(t, d)

    out = pl.pallas_call(
        _moe_dense_body,
        grid=(N_EXP,),
        in_specs=[
            pl.BlockSpec((t, d), lambda e: (0, 0)),
            pl.BlockSpec((d, N_EXP), lambda e: (0, 0)),
            pl.BlockSpec((1, d, f), lambda e: (e, 0, 0)),
            pl.BlockSpec((1, 1, f), lambda e: (e, 0, 0)),
            pl.BlockSpec((1, f, d), lambda e: (e, 0, 0)),
            pl.BlockSpec((1, 1, d), lambda e: (e, 0, 0)),
        ],
        out_specs=pl.BlockSpec((t, d), lambda e: (0, 0)),
        out_shape=jax.ShapeDtypeStruct((t, d), jnp.float32),
        scratch_shapes=[
            pltpu.VMEM((t, N_EXP), jnp.float32),
        ],
        compiler_params=pltpu.CompilerParams(
            dimension_semantics=("arbitrary",),
        ),
    )(flat, w_router, w1, b1.reshape(N_EXP, 1, f), w2, b2.reshape(N_EXP, 1, d))
    return out.reshape(b, s, d)
